# Initial kernel scaffold; baseline (speedup 1.0000x reference)
import jax
import jax.numpy as jnp
from jax.experimental import pallas as pl


def kernel(node_embedding_matrix, W, b, batch_train_x_index):
    raise NotImplementedError("write your pallas kernel here")


# speedup vs baseline: 4.3277x; 4.3277x over previous
import functools

import jax
import jax.numpy as jnp
from jax import lax
from jax.experimental import pallas as pl
from jax.experimental.pallas import tpu as pltpu
from jax.experimental.pallas import tpu_sc as plsc

N = 100000
D = 128
L = 16
S = 1024

CHUNK = 128
NFULL = N // CHUNK
TAIL = N - NFULL * CHUNK
NTILES = 32
CHUNKS_PER_TILE = -(-NFULL // NTILES)
SEGS_PER_TILE = S // NTILES


def _search_counts(ids_hbm, targets, pidxbuf, probebuf, psem):
    lo = [jnp.zeros((16,), jnp.int32) for _ in range(4)]
    hi = [jnp.full((16,), N, jnp.int32) for _ in range(4)]

    def it(_, carry):
        lo, hi = carry
        mids = [(l + h) >> 1 for l, h in zip(lo, hi)]
        for k in range(4):
            pidxbuf[pl.ds(16 * k, 16)] = jnp.minimum(mids[k], N - 1)
        pltpu.async_copy(ids_hbm.at[pidxbuf], probebuf, psem).wait()
        nlo, nhi = [], []
        for k in range(4):
            val = probebuf[pl.ds(16 * k, 16)]
            pred = val < targets[k]
            nlo.append(jnp.where(pred, mids[k] + 1, lo[k]))
            nhi.append(jnp.where(pred, hi[k], mids[k]))
        return tuple(nlo), tuple(nhi)

    lo, _ = lax.fori_loop(0, 17, it, (tuple(lo), tuple(hi)))
    return [jnp.minimum(l, N) for l in lo]


def _segsum_body(x_hbm, ids_hbm, sums_out, cnt_out,
                 xbuf, idxbuf, tidxbuf, pidxbuf, probebuf, psem,
                 cntbuf, spm_sum):
    c = lax.axis_index("c")
    s = lax.axis_index("s")
    wid = c * 16 + s

    zv = jnp.zeros((16,), jnp.float32)

    def zrow(r, carry):
        for j in range(D // 16):
            xbuf[r, pl.ds(j * 16, 16)] = zv
        return carry

    lax.fori_loop(0, 64, zrow, 0)
    pltpu.sync_copy(xbuf.at[pl.ds(0, 64)], spm_sum.at[pl.ds(s * 64, 64)])
    plsc.subcore_barrier()

    base = wid * SEGS_PER_TILE
    t0 = base + lax.broadcasted_iota(jnp.int32, (16,), 0)
    t1 = t0 + 16
    lb0, ub0, lb1, ub1 = _search_counts(
        ids_hbm, (t0, t0 + 1, t1, t1 + 1), pidxbuf, probebuf, psem)
    cntbuf[pl.ds(0, 16)] = (ub0 - lb0).astype(jnp.float32)
    cntbuf[pl.ds(16, 16)] = (ub1 - lb1).astype(jnp.float32)
    pltpu.sync_copy(cntbuf, cnt_out.at[pl.ds(base, SEGS_PER_TILE)])

    for i in range(CHUNKS_PER_TILE):
        cid = wid + NTILES * i

        @pl.when(cid < NFULL)
        def _():
            rbase = cid * CHUNK
            pltpu.sync_copy(x_hbm.at[pl.ds(rbase, CHUNK)], xbuf)
            pltpu.sync_copy(ids_hbm.at[pl.ds(rbase, CHUNK)], idxbuf)
            pltpu.sync_copy(xbuf, spm_sum.at[idxbuf], add=True)

    @pl.when(wid == NTILES - 1)
    def _():
        rbase = NFULL * CHUNK
        pltpu.sync_copy(x_hbm.at[pl.ds(rbase, TAIL)], xbuf.at[pl.ds(0, TAIL)])
        pltpu.sync_copy(ids_hbm.at[pl.ds(rbase, TAIL)], tidxbuf)
        pltpu.sync_copy(xbuf.at[pl.ds(0, TAIL)], spm_sum.at[tidxbuf], add=True)

    plsc.subcore_barrier()

    @pl.when(s == 0)
    def _():
        pltpu.sync_copy(spm_sum, sums_out.at[c])


_segsum = functools.partial(
    pl.kernel,
    out_type=(
        jax.ShapeDtypeStruct((2, S, D), jnp.float32),
        jax.ShapeDtypeStruct((S,), jnp.float32),
    ),
    mesh=plsc.VectorSubcoreMesh(core_axis_name="c", subcore_axis_name="s"),
    scratch_types=[
        pltpu.VMEM((CHUNK, D), jnp.float32),
        pltpu.VMEM((CHUNK,), jnp.int32),
        pltpu.VMEM((TAIL,), jnp.int32),
        pltpu.VMEM((64,), jnp.int32),
        pltpu.VMEM((64,), jnp.int32),
        pltpu.SemaphoreType.DMA,
        pltpu.VMEM((SEGS_PER_TILE,), jnp.float32),
        pltpu.VMEM_SHARED((S, D), jnp.float32),
    ],
)(_segsum_body)


def _finalize_body(sums_ref, cnt_ref, w_ref, b_ref, out_ref):
    seg = sums_ref[0] + sums_ref[1]
    cnt = cnt_ref[...]
    m = lax.dot_general(seg, w_ref[...], (((1,), (1,)), ((), ())),
                        preferred_element_type=jnp.float32)
    mean = m / jnp.maximum(cnt, 1.0) + b_ref[...]
    mean = jnp.where(cnt > 0.0, mean, 0.0)
    z = mean - jnp.max(mean, axis=1, keepdims=True)
    out_ref[...] = z - jnp.log(jnp.sum(jnp.exp(z), axis=1, keepdims=True))


def _finalize(sums, cnt, w, b2d):
    return pl.pallas_call(
        _finalize_body,
        out_shape=jax.ShapeDtypeStruct((S, L), jnp.float32),
    )(sums, cnt, w, b2d)


def kernel(node_embedding_matrix, W, b, batch_train_x_index):
    ids32 = batch_train_x_index.astype(jnp.int32)
    sums, cnt = _segsum(node_embedding_matrix, ids32)
    return _finalize(sums, cnt.reshape(S, 1), W, b.reshape(1, L))


# speedup vs baseline: 6.4930x; 1.5003x over previous
import functools

import jax
import jax.numpy as jnp
from jax import lax
from jax.experimental import pallas as pl
from jax.experimental.pallas import tpu as pltpu
from jax.experimental.pallas import tpu_sc as plsc

N = 100000
D = 128
L = 16
S = 1024

SCAT = 128
NFULL = N // SCAT
NTILES = 32
GROUPS_PER_TILE = 25
NGROUPS = NTILES * GROUPS_PER_TILE
NBUF = 6
AHEAD = 4
TRASH = S
SROWS = 1152
ZROWS = SROWS // 16
SEGS_PER_TILE = S // NTILES


def _segsum_body(x_hbm, ids_hbm, idsp_hbm, sums_out, cnt_out,
                 xbuf, idx0, idx1, idx2, idx3, idx4, idx5,
                 pidxbuf, probebuf, psem,
                 rsem0, rsem1, rsem2, rsem3, rsem4, rsem5,
                 ssem0, ssem1, ssem2, ssem3, ssem4, ssem5,
                 cntbuf, spm_sum):
    idxbufs = (idx0, idx1, idx2, idx3, idx4, idx5)
    rsems = (rsem0, rsem1, rsem2, rsem3, rsem4, rsem5)
    ssems = (ssem0, ssem1, ssem2, ssem3, ssem4, ssem5)
    c = lax.axis_index("c")
    s = lax.axis_index("s")
    wid = c * 16 + s

    zv = jnp.zeros((16,), jnp.float32)

    def zrow(r, carry):
        for j in range(D // 16):
            xbuf[0, r, pl.ds(j * 16, 16)] = zv
        return carry

    lax.fori_loop(0, ZROWS, zrow, 0)
    pltpu.sync_copy(xbuf.at[0, pl.ds(0, ZROWS)],
                    spm_sum.at[pl.ds(s * ZROWS, ZROWS)])
    plsc.subcore_barrier()

    base = wid * SEGS_PER_TILE
    t0 = base + lax.broadcasted_iota(jnp.int32, (16,), 0)
    t1 = t0 + 16
    targets = (t0, t0 + 1, t1, t1 + 1)
    lo = [jnp.zeros((16,), jnp.int32) for _ in range(4)]
    hi = [jnp.full((16,), N, jnp.int32) for _ in range(4)]

    def probe_start(lo, hi):
        mids = [(l + h) >> 1 for l, h in zip(lo, hi)]
        for k in range(4):
            pidxbuf[pl.ds(16 * k, 16)] = jnp.minimum(mids[k], N - 1)
        return mids, pltpu.async_copy(ids_hbm.at[pidxbuf], probebuf, psem)

    def probe_finish(lo, hi, mids, desc):
        desc.wait()
        nlo, nhi = [], []
        for k in range(4):
            val = probebuf[pl.ds(16 * k, 16)]
            pred = val < targets[k]
            nlo.append(jnp.where(pred, mids[k] + 1, lo[k]))
            nhi.append(jnp.where(pred, hi[k], mids[k]))
        return nlo, nhi

    rdescs = {}
    sdescs = {}

    def issue_group(i):
        slot = i % NBUF
        g = wid * GROUPS_PER_TILE + i
        off = pl.multiple_of(jnp.minimum(g * SCAT, N - SCAT), 8)
        dx = pltpu.async_copy(x_hbm.at[pl.ds(off, SCAT)], xbuf.at[slot],
                              rsems[slot])
        di = pltpu.async_copy(idsp_hbm.at[pl.ds(g * SCAT, SCAT)],
                              idxbufs[slot], rsems[slot])
        rdescs[i] = (dx, di)

    for i in range(min(AHEAD, GROUPS_PER_TILE)):
        issue_group(i)

    for i in range(GROUPS_PER_TILE):
        slot = i % NBUF
        if 1 <= i <= 17:
            lo, hi = probe_finish(lo, hi, mids, pdesc)
        if i < 17:
            mids, pdesc = probe_start(lo, hi)
        if i + AHEAD - NBUF >= 0:
            sdescs[i + AHEAD - NBUF].wait()
        if i + AHEAD < GROUPS_PER_TILE:
            issue_group(i + AHEAD)
        dx, di = rdescs[i]
        dx.wait()
        di.wait()
        sdescs[i] = pltpu.async_copy(xbuf.at[slot],
                                     spm_sum.at[idxbufs[slot]],
                                     ssems[slot], add=True)
    for i in range(max(0, GROUPS_PER_TILE - (NBUF - AHEAD)),
                   GROUPS_PER_TILE):
        sdescs[i].wait()

    lo = [jnp.minimum(l, N) for l in lo]
    cntbuf[pl.ds(0, 16)] = (lo[1] - lo[0]).astype(jnp.float32)
    cntbuf[pl.ds(16, 16)] = (lo[3] - lo[2]).astype(jnp.float32)
    pltpu.sync_copy(cntbuf, cnt_out.at[pl.ds(base, SEGS_PER_TILE)])

    plsc.subcore_barrier()

    pltpu.sync_copy(spm_sum.at[pl.ds(s * 64, 64)],
                    sums_out.at[c, pl.ds(s * 64, 64)])


_segsum = functools.partial(
    pl.kernel,
    out_type=(
        jax.ShapeDtypeStruct((2, S, D), jnp.float32),
        jax.ShapeDtypeStruct((S,), jnp.float32),
    ),
    mesh=plsc.VectorSubcoreMesh(core_axis_name="c", subcore_axis_name="s"),
    scratch_types=[
        pltpu.VMEM((NBUF, SCAT, D), jnp.float32),
    ] + [pltpu.VMEM((SCAT,), jnp.int32)] * NBUF
    + [
        pltpu.VMEM((64,), jnp.int32),
        pltpu.VMEM((64,), jnp.int32),
        pltpu.SemaphoreType.DMA,
    ] + [pltpu.SemaphoreType.DMA] * NBUF
    + [pltpu.SemaphoreType.DMA] * NBUF
    + [
        pltpu.VMEM((SEGS_PER_TILE,), jnp.float32),
        pltpu.VMEM_SHARED((SROWS, D), jnp.float32),
    ],
)(_segsum_body)


def _finalize_body(sums_ref, cnt_ref, w_ref, b_ref, out_ref):
    seg = sums_ref[0] + sums_ref[1]
    cnt = cnt_ref[...]
    m = lax.dot_general(seg, w_ref[...], (((1,), (1,)), ((), ())),
                        preferred_element_type=jnp.float32)
    mean = m / jnp.maximum(cnt, 1.0) + b_ref[...]
    mean = jnp.where(cnt > 0.0, mean, 0.0)
    z = mean - jnp.max(mean, axis=1, keepdims=True)
    out_ref[...] = z - jnp.log(jnp.sum(jnp.exp(z), axis=1, keepdims=True))


def _finalize(sums, cnt, w, b2d):
    return pl.pallas_call(
        _finalize_body,
        out_shape=jax.ShapeDtypeStruct((S, L), jnp.float32),
    )(sums, cnt, w, b2d)


def kernel(node_embedding_matrix, W, b, batch_train_x_index):
    ids32 = batch_train_x_index.astype(jnp.int32)
    nfloor = NFULL * SCAT
    idsp = jnp.concatenate([
        ids32[:nfloor],
        jnp.full((SCAT - (N - nfloor),), TRASH, jnp.int32),
        ids32[nfloor:],
        jnp.full((NGROUPS * SCAT - N - (SCAT - (N - nfloor)),), TRASH,
                 jnp.int32),
    ])
    sums, cnt = _segsum(node_embedding_matrix, ids32, idsp)
    return _finalize(sums, cnt.reshape(S, 1), W, b.reshape(1, L))
